# baseline (speedup 1.0000x reference)
import functools
import math

import jax
import jax.numpy as jnp
from jax import lax
from jax.experimental import pallas as pl
from jax.experimental.pallas import tpu as pltpu
from jax.experimental.pallas import tpu_sc as plsc

D_MODEL = 64
D_PAD = 128
S_LEN = 200
BL = 128
SCALE = math.sqrt(D_MODEL)
LANES = 16

_INFO = plsc.get_sparse_core_info()
_NC = _INFO.num_cores
_NS = _INFO.num_subcores
_NW = _NC * _NS


@functools.lru_cache(maxsize=None)
def _build_gather(n_b: int, vocab: int):
    assert n_b == _NW * BL
    assert S_LEN % 2 == 0
    half = S_LEN // 2

    mesh = plsc.VectorSubcoreMesh(core_axis_name="c", subcore_axis_name="s")

    @functools.partial(
        pl.kernel,
        mesh=mesh,
        out_type=jax.ShapeDtypeStruct((S_LEN, D_MODEL, n_b), jnp.float32),
        scratch_types=[
            pltpu.VMEM((S_LEN * BL,), jnp.int32),
            pltpu.VMEM((BL, D_PAD), jnp.float32),
            pltpu.VMEM((BL, D_PAD), jnp.float32),
            pltpu.VMEM((D_MODEL, BL), jnp.float32),
            pltpu.VMEM((D_MODEL, BL), jnp.float32),
            pltpu.SemaphoreType.DMA,
            pltpu.SemaphoreType.DMA,
            pltpu.SemaphoreType.DMA,
            pltpu.SemaphoreType.DMA,
        ],
        compiler_params=pltpu.CompilerParams(needs_layout_passes=False),
    )
    def gather_kernel(idx_hbm, table_hbm, out_hbm, ibuf,
                      ga, gb, ta, tb, sga, sgb, swa, swb):
        wid = lax.axis_index("s") * _NC + lax.axis_index("c")

        pltpu.sync_copy(idx_hbm.at[pl.ds(wid * (S_LEN * BL), S_LEN * BL)],
                        ibuf)

        def fire_gather(s, g, sem):
            pltpu.async_copy(table_hbm.at[ibuf.at[pl.ds(s * BL, BL)]],
                             g, sem)

        def wait_gather(g, sem):
            pltpu.make_async_copy(table_hbm.at[pl.ds(0, BL)], g, sem).wait()

        def transpose_scale(g, o):
            for i in range(BL // LANES):
                rows = lax.iota(jnp.int32, LANES) + i * LANES
                for c in range(D_MODEL):
                    cols = jnp.full((LANES,), c, jnp.int32)
                    v = plsc.load_gather(g, [rows, cols])
                    o[c, pl.ds(i * LANES, LANES)] = v * SCALE

        def fire_write(s, o, sem):
            pltpu.async_copy(o, out_hbm.at[s, :, pl.ds(wid * BL, BL)], sem)

        def wait_write(o, sem):
            pltpu.make_async_copy(table_hbm.at[pl.ds(0, D_MODEL)], o,
                                  sem).wait()

        fire_gather(0, ga, sga)

        def loop_body(t, carry):
            s0 = 2 * t
            s1 = 2 * t + 1
            fire_gather(s1, gb, sgb)
            wait_gather(ga, sga)

            @pl.when(t > 0)
            def _():
                wait_write(ta, swa)

            transpose_scale(ga, ta)
            fire_write(s0, ta, swa)

            @pl.when(t < half - 1)
            def _():
                fire_gather(s0 + 2, ga, sga)

            wait_gather(gb, sgb)

            @pl.when(t > 0)
            def _():
                wait_write(tb, swb)

            transpose_scale(gb, tb)
            fire_write(s1, tb, swb)
            return carry

        lax.fori_loop(0, half, loop_body, 0)

        wait_write(ta, swa)
        wait_write(tb, swb)

    return gather_kernel


def kernel(x, table):
    n_b, s = x.shape
    vocab, d = table.shape
    assert d == D_MODEL and s == S_LEN and n_b == _NW * BL
    xw = (x.astype(jnp.int32)
          .reshape(_NW, BL, S_LEN)
          .transpose(0, 2, 1)
          .reshape(-1))
    tpad = jnp.pad(table, ((0, 0), (0, D_PAD - D_MODEL)))
    outT = _build_gather(n_b, vocab)(xw, tpad)
    return outT.transpose(2, 0, 1)


# speedup vs baseline: 1.7378x; 1.7378x over previous
import functools
import math

import jax
import jax.numpy as jnp
from jax import lax
from jax.experimental import pallas as pl
from jax.experimental.pallas import tpu as pltpu
from jax.experimental.pallas import tpu_sc as plsc

D_MODEL = 64
D_PAD = 128
S_LEN = 200
BL = 128
SCALE = math.sqrt(D_MODEL)
LANES = 16

_INFO = plsc.get_sparse_core_info()
_NC = _INFO.num_cores
_NS = _INFO.num_subcores
_NW = _NC * _NS


@functools.lru_cache(maxsize=None)
def _build_gather(n_b: int, vocab: int):
    assert n_b == _NW * BL
    assert S_LEN % 2 == 0
    half = S_LEN // 2

    mesh = plsc.VectorSubcoreMesh(core_axis_name="c", subcore_axis_name="s")

    @functools.partial(
        pl.kernel,
        mesh=mesh,
        out_type=jax.ShapeDtypeStruct((S_LEN, D_MODEL, n_b), jnp.float32),
        scratch_types=[
            pltpu.VMEM((S_LEN * BL,), jnp.int32),
            pltpu.VMEM((BL, D_PAD), jnp.float32),
            pltpu.VMEM((BL, D_PAD), jnp.float32),
            pltpu.VMEM((D_MODEL, BL), jnp.float32),
            pltpu.VMEM((D_MODEL, BL), jnp.float32),
            pltpu.SemaphoreType.DMA,
            pltpu.SemaphoreType.DMA,
            pltpu.SemaphoreType.DMA,
            pltpu.SemaphoreType.DMA,
        ],
        compiler_params=pltpu.CompilerParams(needs_layout_passes=False),
    )
    def gather_kernel(idx_hbm, table_hbm, out_hbm, ibuf,
                      ga, gb, ta, tb, sga, sgb, swa, swb):
        wid = lax.axis_index("s") * _NC + lax.axis_index("c")

        pltpu.sync_copy(idx_hbm.at[pl.ds(wid * (S_LEN * BL), S_LEN * BL)],
                        ibuf)

        def fire_gather(s, g, sem):
            pltpu.async_copy(table_hbm.at[ibuf.at[pl.ds(s * BL, BL)]],
                             g, sem)

        def wait_gather(g, sem):
            pltpu.make_async_copy(table_hbm.at[pl.ds(0, BL)], g, sem).wait()

        def transpose_scale(g, o):
            base_rows = lax.iota(jnp.int32, LANES)

            @plsc.parallel_loop(0, D_MODEL, unroll=8)
            def _(c):
                cols = jnp.full((LANES,), c, jnp.int32)
                for i in range(BL // LANES):
                    v = plsc.load_gather(g, [base_rows + i * LANES, cols])
                    o[c, pl.ds(i * LANES, LANES)] = v * SCALE

        def fire_write(s, o, sem):
            pltpu.async_copy(o, out_hbm.at[s, :, pl.ds(wid * BL, BL)], sem)

        def wait_write(o, sem):
            pltpu.make_async_copy(table_hbm.at[pl.ds(0, D_MODEL)], o,
                                  sem).wait()

        fire_gather(0, ga, sga)

        def loop_body(t, carry):
            s0 = 2 * t
            s1 = 2 * t + 1
            fire_gather(s1, gb, sgb)
            wait_gather(ga, sga)

            @pl.when(t > 0)
            def _():
                wait_write(ta, swa)

            transpose_scale(ga, ta)
            fire_write(s0, ta, swa)

            @pl.when(t < half - 1)
            def _():
                fire_gather(s0 + 2, ga, sga)

            wait_gather(gb, sgb)

            @pl.when(t > 0)
            def _():
                wait_write(tb, swb)

            transpose_scale(gb, tb)
            fire_write(s1, tb, swb)
            return carry

        lax.fori_loop(0, half, loop_body, 0)

        wait_write(ta, swa)
        wait_write(tb, swb)

    return gather_kernel


def kernel(x, table):
    n_b, s = x.shape
    vocab, d = table.shape
    assert d == D_MODEL and s == S_LEN and n_b == _NW * BL
    xw = (x.astype(jnp.int32)
          .reshape(_NW, BL, S_LEN)
          .transpose(0, 2, 1)
          .reshape(-1))
    tpad = jnp.pad(table, ((0, 0), (0, D_PAD - D_MODEL)))
    outT = _build_gather(n_b, vocab)(xw, tpad)
    return outT.transpose(2, 0, 1)


# speedup vs baseline: 1.7439x; 1.0035x over previous
import functools
import math

import jax
import jax.numpy as jnp
from jax import lax
from jax.experimental import pallas as pl
from jax.experimental.pallas import tpu as pltpu
from jax.experimental.pallas import tpu_sc as plsc

D_MODEL = 64
D_PAD = 128
S_LEN = 200
BL = 128
CH = 2
SCALE = math.sqrt(D_MODEL)
LANES = 16

_INFO = plsc.get_sparse_core_info()
_NC = _INFO.num_cores
_NS = _INFO.num_subcores
_NW = _NC * _NS


@functools.lru_cache(maxsize=None)
def _build_gather(n_b: int, vocab: int):
    assert n_b == _NW * BL
    assert S_LEN % (2 * CH) == 0
    half = S_LEN // (2 * CH)

    mesh = plsc.VectorSubcoreMesh(core_axis_name="c", subcore_axis_name="s")

    @functools.partial(
        pl.kernel,
        mesh=mesh,
        out_type=jax.ShapeDtypeStruct((S_LEN, D_MODEL, n_b), jnp.float32),
        scratch_types=[
            pltpu.VMEM((S_LEN * BL,), jnp.int32),
            pltpu.VMEM((CH * BL, D_PAD), jnp.float32),
            pltpu.VMEM((CH * BL, D_PAD), jnp.float32),
            pltpu.VMEM((CH * D_MODEL, BL), jnp.float32),
            pltpu.VMEM((CH * D_MODEL, BL), jnp.float32),
            pltpu.SemaphoreType.DMA,
            pltpu.SemaphoreType.DMA,
            pltpu.SemaphoreType.DMA,
            pltpu.SemaphoreType.DMA,
        ],
        compiler_params=pltpu.CompilerParams(needs_layout_passes=False),
    )
    def gather_kernel(idx_hbm, table_hbm, out_hbm, ibuf,
                      ga, gb, ta, tb, sga, sgb, swa, swb):
        wid = lax.axis_index("s") * _NC + lax.axis_index("c")

        pltpu.sync_copy(idx_hbm.at[pl.ds(wid * (S_LEN * BL), S_LEN * BL)],
                        ibuf)

        def fire_gather(c, g, sem):
            for p in range(CH):
                pltpu.async_copy(
                    table_hbm.at[ibuf.at[pl.ds((c * CH + p) * BL, BL)]],
                    g.at[pl.ds(p * BL, BL), :], sem)

        def wait_gather(g, sem):
            pltpu.make_async_copy(table_hbm.at[pl.ds(0, CH * BL)], g,
                                  sem).wait()

        def transpose_scale(g, o):
            base_rows = lax.iota(jnp.int32, LANES)

            @plsc.parallel_loop(0, CH * D_MODEL, unroll=8)
            def _(c2):
                c = lax.rem(c2, D_MODEL)
                roff = (c2 // D_MODEL) * BL
                cols = jnp.full((LANES,), c, jnp.int32)
                for i in range(BL // LANES):
                    v = plsc.load_gather(g, [base_rows + roff + i * LANES,
                                             cols])
                    o[c2, pl.ds(i * LANES, LANES)] = v * SCALE

        def fire_write(c, o, sem):
            for p in range(CH):
                pltpu.async_copy(o.at[pl.ds(p * D_MODEL, D_MODEL), :],
                                 out_hbm.at[c * CH + p, :,
                                            pl.ds(wid * BL, BL)], sem)

        def wait_write(o, sem):
            pltpu.make_async_copy(table_hbm.at[pl.ds(0, CH * D_MODEL)], o,
                                  sem).wait()

        fire_gather(0, ga, sga)

        def loop_body(t, carry):
            s0 = 2 * t
            s1 = 2 * t + 1
            fire_gather(s1, gb, sgb)
            wait_gather(ga, sga)

            @pl.when(t > 0)
            def _():
                wait_write(ta, swa)

            transpose_scale(ga, ta)
            fire_write(s0, ta, swa)

            @pl.when(t < half - 1)
            def _():
                fire_gather(s0 + 2, ga, sga)

            wait_gather(gb, sgb)

            @pl.when(t > 0)
            def _():
                wait_write(tb, swb)

            transpose_scale(gb, tb)
            fire_write(s1, tb, swb)
            return carry

        lax.fori_loop(0, half, loop_body, 0)

        wait_write(ta, swa)
        wait_write(tb, swb)

    return gather_kernel


def kernel(x, table):
    n_b, s = x.shape
    vocab, d = table.shape
    assert d == D_MODEL and s == S_LEN and n_b == _NW * BL
    xw = (x.astype(jnp.int32)
          .reshape(_NW, BL, S_LEN)
          .transpose(0, 2, 1)
          .reshape(-1))
    tpad = jnp.pad(table, ((0, 0), (0, D_PAD - D_MODEL)))
    outT = _build_gather(n_b, vocab)(xw, tpad)
    return outT.transpose(2, 0, 1)
